# baseline (speedup 1.0000x reference)
import jax
import jax.numpy as jnp
from jax.experimental import pallas as pl
from jax.experimental.pallas import tpu as pltpu

_CH = 512
_DEPTH = 4


def _stream_kernel(x_hbm, pos_hbm, o_hbm, pos_vmem, xbuf, obuf,
                   in_sems, pos_sems, out_sems):
    n_rows = x_hbm.shape[0]
    n_chunks = n_rows // _CH
    n_pos_chunks = pos_hbm.shape[0] // _CH

    def in_copy(i, slot):
        return pltpu.make_async_copy(
            x_hbm.at[pl.ds(i * _CH, _CH)], xbuf.at[slot], in_sems.at[slot])

    def pos_copy(i):
        return pltpu.make_async_copy(
            pos_hbm.at[pl.ds(i * _CH, _CH)], pos_vmem.at[i], pos_sems.at[i])

    def out_copy(i, slot):
        return pltpu.make_async_copy(
            obuf.at[slot], o_hbm.at[pl.ds(i * _CH, _CH)], out_sems.at[slot])

    for k in range(_DEPTH):
        in_copy(k, k).start()
        pos_copy(k).start()

    def body(i, _):
        slot = jax.lax.rem(i, _DEPTH)
        in_copy(i, slot).wait()
        pos_chunk = jax.lax.rem(i, n_pos_chunks)

        @pl.when(i < n_pos_chunks)
        def _():
            pos_copy(i).wait()

        @pl.when(i >= _DEPTH)
        def _():
            out_copy(i - _DEPTH, slot).wait()

        obuf[slot] = xbuf[slot] + pos_vmem[pos_chunk]
        out_copy(i, slot).start()

        nxt = i + _DEPTH

        @pl.when(nxt < n_chunks)
        def _():
            in_copy(nxt, jax.lax.rem(nxt, _DEPTH)).start()

        @pl.when(nxt < n_pos_chunks)
        def _():
            pos_copy(nxt).start()

        return 0

    jax.lax.fori_loop(0, n_chunks, body, 0)

    for k in range(_DEPTH):
        out_copy(n_chunks - _DEPTH + k,
                 jax.lax.rem(n_chunks - _DEPTH + k, _DEPTH)).wait()


def kernel(x, pos_table):
    B, T, D = x.shape
    x2 = x.reshape(B * T, D)
    n_pos_chunks = T // _CH
    out = pl.pallas_call(
        _stream_kernel,
        in_specs=[
            pl.BlockSpec(memory_space=pltpu.MemorySpace.HBM),
            pl.BlockSpec(memory_space=pltpu.MemorySpace.HBM),
        ],
        out_specs=pl.BlockSpec(memory_space=pltpu.MemorySpace.HBM),
        out_shape=jax.ShapeDtypeStruct((B * T, D), x.dtype),
        scratch_shapes=[
            pltpu.VMEM((n_pos_chunks, _CH, D), jnp.float32),
            pltpu.VMEM((_DEPTH, _CH, D), jnp.float32),
            pltpu.VMEM((_DEPTH, _CH, D), jnp.float32),
            pltpu.SemaphoreType.DMA((_DEPTH,)),
            pltpu.SemaphoreType.DMA((n_pos_chunks,)),
            pltpu.SemaphoreType.DMA((_DEPTH,)),
        ],
    )(x2, pos_table)
    return out.reshape(B, T, D)


# speedup vs baseline: 1.0030x; 1.0030x over previous
import jax
import jax.numpy as jnp
from jax.experimental import pallas as pl
from jax.experimental.pallas import tpu as pltpu

_CH = 256
_DEPTH = 8


def _stream_kernel(x_hbm, pos_hbm, o_hbm, pos_vmem, xbuf, obuf,
                   in_sems, pos_sems, out_sems):
    n_rows = x_hbm.shape[0]
    n_chunks = n_rows // _CH
    n_pos_chunks = pos_hbm.shape[0] // _CH

    def in_copy(i, slot):
        return pltpu.make_async_copy(
            x_hbm.at[pl.ds(i * _CH, _CH)], xbuf.at[slot], in_sems.at[slot])

    def pos_copy(i):
        return pltpu.make_async_copy(
            pos_hbm.at[pl.ds(i * _CH, _CH)], pos_vmem.at[i], pos_sems.at[i])

    def out_copy(i, slot):
        return pltpu.make_async_copy(
            obuf.at[slot], o_hbm.at[pl.ds(i * _CH, _CH)], out_sems.at[slot])

    for k in range(_DEPTH):
        in_copy(k, k).start()
        pos_copy(k).start()

    def body(i, _):
        slot = jax.lax.rem(i, _DEPTH)
        in_copy(i, slot).wait()
        pos_chunk = jax.lax.rem(i, n_pos_chunks)

        @pl.when(i < n_pos_chunks)
        def _():
            pos_copy(i).wait()

        @pl.when(i >= _DEPTH)
        def _():
            out_copy(i - _DEPTH, slot).wait()

        obuf[slot] = xbuf[slot] + pos_vmem[pos_chunk]
        out_copy(i, slot).start()

        nxt = i + _DEPTH

        @pl.when(nxt < n_chunks)
        def _():
            in_copy(nxt, jax.lax.rem(nxt, _DEPTH)).start()

        @pl.when(nxt < n_pos_chunks)
        def _():
            pos_copy(nxt).start()

        return 0

    jax.lax.fori_loop(0, n_chunks, body, 0)

    for k in range(_DEPTH):
        out_copy(n_chunks - _DEPTH + k,
                 jax.lax.rem(n_chunks - _DEPTH + k, _DEPTH)).wait()


def kernel(x, pos_table):
    B, T, D = x.shape
    x2 = x.reshape(B * T, D)
    n_pos_chunks = T // _CH
    out = pl.pallas_call(
        _stream_kernel,
        in_specs=[
            pl.BlockSpec(memory_space=pltpu.MemorySpace.HBM),
            pl.BlockSpec(memory_space=pltpu.MemorySpace.HBM),
        ],
        out_specs=pl.BlockSpec(memory_space=pltpu.MemorySpace.HBM),
        out_shape=jax.ShapeDtypeStruct((B * T, D), x.dtype),
        scratch_shapes=[
            pltpu.VMEM((n_pos_chunks, _CH, D), jnp.float32),
            pltpu.VMEM((_DEPTH, _CH, D), jnp.float32),
            pltpu.VMEM((_DEPTH, _CH, D), jnp.float32),
            pltpu.SemaphoreType.DMA((_DEPTH,)),
            pltpu.SemaphoreType.DMA((n_pos_chunks,)),
            pltpu.SemaphoreType.DMA((_DEPTH,)),
        ],
    )(x2, pos_table)
    return out.reshape(B, T, D)
